# Initial kernel scaffold; baseline (speedup 1.0000x reference)
import jax
import jax.numpy as jnp
from jax.experimental import pallas as pl


def kernel(x_emb, z, wx, wz, wh, b, wout, bout):
    raise NotImplementedError("write your pallas kernel here")


# speedup vs baseline: 3.2412x; 3.2412x over previous
import functools

import jax
import jax.numpy as jnp
from jax.experimental import pallas as pl
from jax.experimental.pallas import tpu as pltpu


def _lstm_fused_kernel(
    x_ref,
    zb_ref,
    wx_ref,
    wh_ref,
    wout_ref,
    bout_ref,
    out_ref,
    h_scr,
    c_scr,
    hall_scr,
    xp_scr,
    *, tt,
):
    vb = pl.program_id(1)
    tb = pl.program_id(2)
    Bb, _, E = x_ref.shape
    H = h_scr.shape[1]

    @pl.when(vb == 0)
    def _recurrence():
        @pl.when(tb == 0)
        def _init():
            h_scr[...] = jnp.zeros_like(h_scr)
            c_scr[...] = jnp.zeros_like(c_scr)

        xp = jnp.dot(x_ref[...].reshape(Bb * tt, E), wx_ref[...],
                     preferred_element_type=jnp.float32)
        xp_scr[...] = xp.reshape(Bb, tt, 4 * H) + zb_ref[...][:, None, :]

        h = h_scr[...]
        c = c_scr[...]
        hs = []
        for s in range(tt):
            gates = xp_scr[:, s, :] + jnp.dot(
                h, wh_ref[...], preferred_element_type=jnp.float32)
            ifo = jax.nn.sigmoid(gates[:, :3 * H])
            g = jnp.tanh(gates[:, 3 * H:])
            c = ifo[:, H:2 * H] * c + ifo[:, :H] * g
            h = (ifo[:, 2 * H:] * jnp.tanh(c)).astype(jnp.bfloat16)
            hs.append(h)
            if s % 2 == 1:
                hall_scr[:, tb, s - 1:s + 1, :] = jnp.stack(hs[-2:], axis=1)
        h_scr[...] = h
        c_scr[...] = c

    Vt = wout_ref.shape[1]
    hblk = hall_scr[:, tb].reshape(Bb * tt, H)
    logits = jnp.dot(hblk, wout_ref[...],
                     preferred_element_type=jnp.float32) + bout_ref[...]
    out_ref[...] = logits.reshape(Bb, tt, Vt)


def kernel(x_emb, z, wx, wz, wh, b, wout, bout):
    B, T, E = x_emb.shape
    H = wh.shape[0]
    V = bout.shape[-1]

    n_bh = 2 if B % 16 == 0 else 1
    Bb = B // n_bh
    tt = 16 if T % 16 == 0 else T
    n_tb = T // tt
    Vt = 2048 if V % 2048 == 0 else V
    n_vb = V // Vt

    zb = jnp.dot(z, wz) + b

    xb = x_emb.astype(jnp.bfloat16)
    wxb = wx.astype(jnp.bfloat16)
    whb = wh.astype(jnp.bfloat16)
    woutb = wout.astype(jnp.bfloat16)

    out = pl.pallas_call(
        functools.partial(_lstm_fused_kernel, tt=tt),
        grid=(n_bh, n_vb, n_tb),
        in_specs=[
            pl.BlockSpec((Bb, tt, E), lambda bh, vb, tb: (bh, tb, 0)),
            pl.BlockSpec((Bb, 4 * H), lambda bh, vb, tb: (bh, 0)),
            pl.BlockSpec((E, 4 * H), lambda bh, vb, tb: (0, 0)),
            pl.BlockSpec((H, 4 * H), lambda bh, vb, tb: (0, 0)),
            pl.BlockSpec((H, Vt), lambda bh, vb, tb: (0, vb)),
            pl.BlockSpec((1, Vt), lambda bh, vb, tb: (0, vb)),
        ],
        out_specs=pl.BlockSpec((Bb, tt, Vt), lambda bh, vb, tb: (bh, tb, vb)),
        out_shape=jax.ShapeDtypeStruct((B, T, V), jnp.float32),
        scratch_shapes=[
            pltpu.VMEM((Bb, H), jnp.bfloat16),
            pltpu.VMEM((Bb, H), jnp.float32),
            pltpu.VMEM((Bb, n_tb, tt, H), jnp.bfloat16),
            pltpu.VMEM((Bb, tt, 4 * H), jnp.float32),
        ],
        compiler_params=pltpu.CompilerParams(
            dimension_semantics=("parallel", "arbitrary", "arbitrary"),
        ),
    )(xb, zb, wxb, whb, woutb, bout)
    return out


# speedup vs baseline: 3.5894x; 1.1074x over previous
import functools

import jax
import jax.numpy as jnp
from jax.experimental import pallas as pl
from jax.experimental.pallas import tpu as pltpu


def _lstm_fused_kernel(
    x_ref,
    zb_ref,
    wx_ref,
    wh_ref,
    wout_ref,
    bout_ref,
    out_ref,
    h_scr,
    c_scr,
    htmp_scr,
    hall_scr,
    xp_scr,
    *, tt,
):
    vb = pl.program_id(1)
    tb = pl.program_id(2)
    _, Bb, E = x_ref.shape
    H = h_scr.shape[1]

    @pl.when(vb == 0)
    def _recurrence():
        @pl.when(tb == 0)
        def _init():
            h_scr[...] = jnp.zeros_like(h_scr)
            c_scr[...] = jnp.zeros_like(c_scr)

        xp = jnp.dot(x_ref[...].reshape(tt * Bb, E), wx_ref[...],
                     preferred_element_type=jnp.float32)
        xp_scr[...] = xp.reshape(tt, Bb, 4 * H) + zb_ref[...][None]

        h = h_scr[...]
        c = c_scr[...]
        for s in range(tt):
            gates = xp_scr[s] + jnp.dot(
                h, wh_ref[...], preferred_element_type=jnp.float32)
            ifo = jax.nn.sigmoid(gates[:, :3 * H])
            g = jnp.tanh(gates[:, 3 * H:])
            c = ifo[:, H:2 * H] * c + ifo[:, :H] * g
            h = (ifo[:, 2 * H:] * jnp.tanh(c)).astype(jnp.bfloat16)
            htmp_scr[s] = h
        h_scr[...] = h
        c_scr[...] = c
        hall_scr[:, tb] = jnp.transpose(htmp_scr[...], (1, 0, 2))

    Vt = wout_ref.shape[1]
    hblk = hall_scr[:, tb].reshape(Bb * tt, H)
    logits = jnp.dot(hblk, wout_ref[...],
                     preferred_element_type=jnp.float32) + bout_ref[...]
    out_ref[...] = logits.reshape(Bb, tt, Vt)


def kernel(x_emb, z, wx, wz, wh, b, wout, bout):
    B, T, E = x_emb.shape
    H = wh.shape[0]
    V = bout.shape[-1]

    n_bh = 2 if B % 16 == 0 else 1
    Bb = B // n_bh
    tt = 16 if T % 16 == 0 else T
    n_tb = T // tt
    Vt = 2048 if V % 2048 == 0 else V
    n_vb = V // Vt

    zb = jnp.dot(z, wz) + b

    x_tm = jnp.transpose(x_emb, (1, 0, 2)).astype(jnp.bfloat16)
    wxb = wx.astype(jnp.bfloat16)
    whb = wh.astype(jnp.bfloat16)
    woutb = wout.astype(jnp.bfloat16)

    out = pl.pallas_call(
        functools.partial(_lstm_fused_kernel, tt=tt),
        grid=(n_bh, n_vb, n_tb),
        in_specs=[
            pl.BlockSpec((tt, Bb, E), lambda bh, vb, tb: (tb, bh, 0)),
            pl.BlockSpec((Bb, 4 * H), lambda bh, vb, tb: (bh, 0)),
            pl.BlockSpec((E, 4 * H), lambda bh, vb, tb: (0, 0)),
            pl.BlockSpec((H, 4 * H), lambda bh, vb, tb: (0, 0)),
            pl.BlockSpec((H, Vt), lambda bh, vb, tb: (0, vb)),
            pl.BlockSpec((1, Vt), lambda bh, vb, tb: (0, vb)),
        ],
        out_specs=pl.BlockSpec((Bb, tt, Vt), lambda bh, vb, tb: (bh, tb, vb)),
        out_shape=jax.ShapeDtypeStruct((B, T, V), jnp.float32),
        scratch_shapes=[
            pltpu.VMEM((Bb, H), jnp.bfloat16),
            pltpu.VMEM((Bb, H), jnp.float32),
            pltpu.VMEM((tt, Bb, H), jnp.bfloat16),
            pltpu.VMEM((Bb, n_tb, tt, H), jnp.bfloat16),
            pltpu.VMEM((tt, Bb, 4 * H), jnp.float32),
        ],
        compiler_params=pltpu.CompilerParams(
            dimension_semantics=("parallel", "arbitrary", "arbitrary"),
        ),
    )(x_tm, zb, wxb, whb, woutb, bout)
    return out


# speedup vs baseline: 3.7339x; 1.0403x over previous
import functools

import jax
import jax.numpy as jnp
from jax.experimental import pallas as pl
from jax.experimental.pallas import tpu as pltpu


def _lstm_rec_kernel(
    x_ref,
    zb_ref,
    wx_ref,
    wh_ref,
    hall_ref,
    h_scr,
    c_scr,
    htmp_scr,
    xp_scr,
    *, tt,
):
    tb = pl.program_id(1)
    _, Bb, E = x_ref.shape
    H = h_scr.shape[1]

    @pl.when(tb == 0)
    def _init():
        h_scr[...] = jnp.zeros_like(h_scr)
        c_scr[...] = jnp.zeros_like(c_scr)

    xp = jnp.dot(x_ref[...].reshape(tt * Bb, E), wx_ref[...],
                 preferred_element_type=jnp.float32)
    xp_scr[...] = xp.reshape(tt, Bb, 4 * H) + zb_ref[...][None]

    h = h_scr[...]
    c = c_scr[...]
    for s in range(tt):
        gates = xp_scr[s] + jnp.dot(
            h, wh_ref[...], preferred_element_type=jnp.float32)
        ifo = jax.nn.sigmoid(gates[:, :3 * H])
        g = jnp.tanh(gates[:, 3 * H:])
        c = ifo[:, H:2 * H] * c + ifo[:, :H] * g
        h = (ifo[:, 2 * H:] * jnp.tanh(c)).astype(jnp.bfloat16)
        htmp_scr[s] = h
    h_scr[...] = h
    c_scr[...] = c
    hall_ref[...] = jnp.transpose(htmp_scr[...], (1, 0, 2))


def _proj_kernel(h_ref, w_ref, b_ref, out_ref):
    out_ref[...] = jnp.dot(h_ref[...], w_ref[...],
                           preferred_element_type=jnp.float32) + b_ref[...]


def kernel(x_emb, z, wx, wz, wh, b, wout, bout):
    B, T, E = x_emb.shape
    H = wh.shape[0]
    V = bout.shape[-1]

    n_bh = 2 if B % 16 == 0 else 1
    Bb = B // n_bh
    tt = 16 if T % 16 == 0 else T
    n_tb = T // tt

    zb = jnp.dot(z, wz) + b
    x_tm = jnp.transpose(x_emb, (1, 0, 2)).astype(jnp.bfloat16)
    wxb = wx.astype(jnp.bfloat16)
    whb = wh.astype(jnp.bfloat16)
    woutb = wout.astype(jnp.bfloat16)

    hall = pl.pallas_call(
        functools.partial(_lstm_rec_kernel, tt=tt),
        grid=(n_bh, n_tb),
        in_specs=[
            pl.BlockSpec((tt, Bb, E), lambda bh, tb: (tb, bh, 0)),
            pl.BlockSpec((Bb, 4 * H), lambda bh, tb: (bh, 0)),
            pl.BlockSpec((E, 4 * H), lambda bh, tb: (0, 0)),
            pl.BlockSpec((H, 4 * H), lambda bh, tb: (0, 0)),
        ],
        out_specs=pl.BlockSpec((Bb, tt, H), lambda bh, tb: (bh, tb, 0)),
        out_shape=jax.ShapeDtypeStruct((B, T, H), jnp.bfloat16),
        scratch_shapes=[
            pltpu.VMEM((Bb, H), jnp.bfloat16),
            pltpu.VMEM((Bb, H), jnp.float32),
            pltpu.VMEM((tt, Bb, H), jnp.bfloat16),
            pltpu.VMEM((tt, Bb, 4 * H), jnp.float32),
        ],
        compiler_params=pltpu.CompilerParams(
            dimension_semantics=("parallel", "arbitrary"),
        ),
    )(x_tm, zb, wxb, whb)

    M = B * T
    hflat = hall.reshape(M, H)
    m_half = M // 2
    bm = 1024 if m_half % 1024 == 0 else m_half
    bn = 2048 if V % 2048 == 0 else V
    n_m = m_half // bm
    n_n = V // bn

    out = pl.pallas_call(
        _proj_kernel,
        grid=(2, n_n, n_m),
        in_specs=[
            pl.BlockSpec((bm, H), lambda tc, n, m: (tc * (m_half // bm) + m, 0)),
            pl.BlockSpec((H, bn), lambda tc, n, m: (0, n)),
            pl.BlockSpec((1, bn), lambda tc, n, m: (0, n)),
        ],
        out_specs=pl.BlockSpec((bm, bn), lambda tc, n, m: (tc * (m_half // bm) + m, n)),
        out_shape=jax.ShapeDtypeStruct((M, V), jnp.float32),
        compiler_params=pltpu.CompilerParams(
            dimension_semantics=("parallel", "arbitrary", "arbitrary"),
        ),
    )(hflat, woutb, bout)
    return out.reshape(B, T, V)


# speedup vs baseline: 4.8572x; 1.3009x over previous
import functools

import jax
import jax.numpy as jnp
from jax.experimental import pallas as pl
from jax.experimental.pallas import tpu as pltpu


def _lstm_rec_kernel(
    x_ref,
    zb_ref,
    wx_ref,
    wh_ref,
    hall_ref,
    h_scr,
    c_scr,
    htmp_scr,
    xp_scr,
    *, tt,
):
    tb = pl.program_id(0)
    _, B, E = x_ref.shape
    H = h_scr.shape[1]

    @pl.when(tb == 0)
    def _init():
        h_scr[...] = jnp.zeros_like(h_scr)
        c_scr[...] = jnp.zeros_like(c_scr)

    xp = jnp.dot(x_ref[...].reshape(tt * B, E), wx_ref[...],
                 preferred_element_type=jnp.float32)
    xp_scr[...] = xp.reshape(tt, B, 4 * H) + zb_ref[...][None]

    h = h_scr[...]
    c = c_scr[...]
    for s in range(tt):
        gates = xp_scr[s] + jnp.dot(
            h, wh_ref[...], preferred_element_type=jnp.float32)
        ifo = jax.nn.sigmoid(gates[:, :3 * H])
        g = jnp.tanh(gates[:, 3 * H:])
        c = ifo[:, H:2 * H] * c + ifo[:, :H] * g
        h = (ifo[:, 2 * H:] * jnp.tanh(c)).astype(jnp.bfloat16)
        htmp_scr[s] = h
    h_scr[...] = h
    c_scr[...] = c
    hall_ref[...] = jnp.transpose(htmp_scr[...], (1, 0, 2))


def _proj_kernel(h_ref, w_ref, b_ref, out_ref):
    out_ref[...] = jnp.dot(h_ref[...], w_ref[...],
                           preferred_element_type=jnp.float32) + b_ref[...]


def kernel(x_emb, z, wx, wz, wh, b, wout, bout):
    B, T, E = x_emb.shape
    H = wh.shape[0]
    V = bout.shape[-1]

    tt = 16 if T % 16 == 0 else T
    n_tb = T // tt

    zb = jnp.dot(z, wz) + b
    x_tm = jnp.transpose(x_emb, (1, 0, 2)).astype(jnp.bfloat16)
    wxb = wx.astype(jnp.bfloat16)
    whb = wh.astype(jnp.bfloat16)
    woutb = wout.astype(jnp.bfloat16)

    hall = pl.pallas_call(
        functools.partial(_lstm_rec_kernel, tt=tt),
        grid=(n_tb,),
        in_specs=[
            pl.BlockSpec((tt, B, E), lambda tb: (tb, 0, 0)),
            pl.BlockSpec((B, 4 * H), lambda tb: (0, 0)),
            pl.BlockSpec((E, 4 * H), lambda tb: (0, 0)),
            pl.BlockSpec((H, 4 * H), lambda tb: (0, 0)),
        ],
        out_specs=pl.BlockSpec((B, tt, H), lambda tb: (0, tb, 0)),
        out_shape=jax.ShapeDtypeStruct((B, T, H), jnp.bfloat16),
        scratch_shapes=[
            pltpu.VMEM((B, H), jnp.bfloat16),
            pltpu.VMEM((B, H), jnp.float32),
            pltpu.VMEM((tt, B, H), jnp.bfloat16),
            pltpu.VMEM((tt, B, 4 * H), jnp.float32),
        ],
        compiler_params=pltpu.CompilerParams(
            dimension_semantics=("arbitrary",),
        ),
    )(x_tm, zb, wxb, whb)

    M = B * T
    hflat = hall.reshape(M, H)
    m_half = M // 2
    bm = 1024 if m_half % 1024 == 0 else m_half
    bn = 2048 if V % 2048 == 0 else V
    n_m = m_half // bm
    n_n = V // bn

    out = pl.pallas_call(
        _proj_kernel,
        grid=(2, n_n, n_m),
        in_specs=[
            pl.BlockSpec((bm, H), lambda tc, n, m: (tc * (m_half // bm) + m, 0)),
            pl.BlockSpec((H, bn), lambda tc, n, m: (0, n)),
            pl.BlockSpec((1, bn), lambda tc, n, m: (0, n)),
        ],
        out_specs=pl.BlockSpec((bm, bn), lambda tc, n, m: (tc * (m_half // bm) + m, n)),
        out_shape=jax.ShapeDtypeStruct((M, V), jnp.float32),
        compiler_params=pltpu.CompilerParams(
            dimension_semantics=("parallel", "arbitrary", "arbitrary"),
        ),
    )(hflat, woutb, bout)
    return out.reshape(B, T, V)


# speedup vs baseline: 5.0188x; 1.0333x over previous
import functools

import jax
import jax.numpy as jnp
from jax.experimental import pallas as pl
from jax.experimental.pallas import tpu as pltpu


def _lstm_rec_kernel(
    x_ref,
    zb_ref,
    wx_ref,
    wh_ref,
    hall_ref,
    h_scr,
    c_scr,
    htmp_scr,
    xp_scr,
    *, tt,
):
    tb = pl.program_id(0)
    _, B, E = x_ref.shape
    H = h_scr.shape[1]

    @pl.when(tb == 0)
    def _init():
        h_scr[...] = jnp.zeros_like(h_scr)
        c_scr[...] = jnp.zeros_like(c_scr)

    xp = jnp.dot(x_ref[...].reshape(tt * B, E), wx_ref[...],
                 preferred_element_type=jnp.float32)
    xp_scr[...] = xp.reshape(tt, B, 4 * H) + zb_ref[...][None]

    h = h_scr[...]
    c = c_scr[...]
    for s in range(tt):
        gates = xp_scr[s] + jnp.dot(
            h, wh_ref[...], preferred_element_type=jnp.float32)
        ifo = jax.nn.sigmoid(gates[:, :3 * H])
        g = jnp.tanh(gates[:, 3 * H:])
        c = ifo[:, H:2 * H] * c + ifo[:, :H] * g
        h = (ifo[:, 2 * H:] * jnp.tanh(c)).astype(jnp.bfloat16)
        htmp_scr[s] = h
    h_scr[...] = h
    c_scr[...] = c
    hall_ref[...] = jnp.transpose(htmp_scr[...], (1, 0, 2))


def _proj_kernel(h_ref, w_ref, b_ref, out_ref):
    out_ref[...] = jnp.dot(h_ref[...], w_ref[...],
                           preferred_element_type=jnp.float32) + b_ref[...]


def kernel(x_emb, z, wx, wz, wh, b, wout, bout):
    B, T, E = x_emb.shape
    H = wh.shape[0]
    V = bout.shape[-1]

    tt = 16 if T % 16 == 0 else T
    n_tb = T // tt

    zb = jnp.dot(z, wz) + b
    x_tm = jnp.transpose(x_emb, (1, 0, 2)).astype(jnp.bfloat16)
    wxb = wx.astype(jnp.bfloat16)
    whb = wh.astype(jnp.bfloat16)

    hall = pl.pallas_call(
        functools.partial(_lstm_rec_kernel, tt=tt),
        grid=(n_tb,),
        in_specs=[
            pl.BlockSpec((tt, B, E), lambda tb: (tb, 0, 0)),
            pl.BlockSpec((B, 4 * H), lambda tb: (0, 0)),
            pl.BlockSpec((E, 4 * H), lambda tb: (0, 0)),
            pl.BlockSpec((H, 4 * H), lambda tb: (0, 0)),
        ],
        out_specs=pl.BlockSpec((B, tt, H), lambda tb: (0, tb, 0)),
        out_shape=jax.ShapeDtypeStruct((B, T, H), jnp.bfloat16),
        scratch_shapes=[
            pltpu.VMEM((B, H), jnp.bfloat16),
            pltpu.VMEM((B, H), jnp.float32),
            pltpu.VMEM((tt, B, H), jnp.bfloat16),
            pltpu.VMEM((tt, B, 4 * H), jnp.float32),
        ],
        compiler_params=pltpu.CompilerParams(
            dimension_semantics=("arbitrary",),
        ),
    )(x_tm, zb, wxb, whb)

    M = B * T
    hflat = hall.reshape(M, H)
    bm = 1024 if M % 1024 == 0 else M
    bn = 2048 if V % 2048 == 0 else V
    n_m = M // bm
    n_n = V // bn

    out = pl.pallas_call(
        _proj_kernel,
        grid=(n_n, n_m),
        in_specs=[
            pl.BlockSpec((bm, H), lambda n, m: (m, 0)),
            pl.BlockSpec((H, bn), lambda n, m: (0, n)),
            pl.BlockSpec((1, bn), lambda n, m: (0, n)),
        ],
        out_specs=pl.BlockSpec((bm, bn), lambda n, m: (m, n)),
        out_shape=jax.ShapeDtypeStruct((M, V), jnp.float32),
        compiler_params=pltpu.CompilerParams(
            dimension_semantics=("arbitrary", "arbitrary"),
        ),
    )(hflat, wout, bout)
    return out.reshape(B, T, V)
